# Initial kernel scaffold; baseline (speedup 1.0000x reference)
import jax
import jax.numpy as jnp
from jax.experimental import pallas as pl


def kernel(einds, feats, rels, W1, root1, b1, W2, root2, b2):
    raise NotImplementedError("write your pallas kernel here")


# speedup vs baseline: 13.5001x; 13.5001x over previous
import functools

import jax
import jax.numpy as jnp
from jax import lax
from jax.experimental import pallas as pl
from jax.experimental.pallas import tpu as pltpu
from jax.experimental.pallas import tpu_sc as plsc

N = 10000
E = 320000
R = 8
RN = R * N
NT = 32
EPT = E // NT
CH = 80
NG = EPT // CH
HR = 5120
BN = 400
NB = N // BN


def _xw_body(x_ref, w_ref, o_ref):
    o_ref[...] = jnp.dot(x_ref[...], w_ref[0],
                         preferred_element_type=jnp.float32)[None]


def _xw_tables(x, W):
    r, din, h = W.shape
    return pl.pallas_call(
        _xw_body,
        grid=(r, NB),
        in_specs=[
            pl.BlockSpec((BN, din), lambda j, i: (i, 0)),
            pl.BlockSpec((1, din, h), lambda j, i: (j, 0, 0)),
        ],
        out_specs=pl.BlockSpec((1, BN, h), lambda j, i: (j, i, 0)),
        out_shape=jax.ShapeDtypeStruct((r, N, h), jnp.float32),
    )(x, W)


def _self_body(x_ref, r_ref, b_ref, o_ref):
    o_ref[...] = jnp.dot(x_ref[...], r_ref[...],
                         preferred_element_type=jnp.float32) + b_ref[...]


def _self_term(x, root, b):
    din, h = root.shape
    return pl.pallas_call(
        _self_body,
        grid=(NB,),
        in_specs=[
            pl.BlockSpec((BN, din), lambda i: (i, 0)),
            pl.BlockSpec((din, h), lambda i: (0, 0)),
            pl.BlockSpec((1, h), lambda i: (0, 0)),
        ],
        out_specs=pl.BlockSpec((BN, h), lambda i: (i, 0)),
        out_shape=jax.ShapeDtypeStruct((N, h), jnp.float32),
    )(x, root, b.reshape(1, h))


def _xw2_body(s_ref, p0_ref, p1_ref, w_ref, o_ref):
    h = jnp.maximum(s_ref[...] + p0_ref[...] + p1_ref[...], 0.0)
    o_ref[...] = jnp.dot(h, w_ref[0], preferred_element_type=jnp.float32)[None]


def _xw2_tables(s1, p0, p1, W):
    r, h1, h2 = W.shape
    return pl.pallas_call(
        _xw2_body,
        grid=(r, NB),
        in_specs=[
            pl.BlockSpec((BN, h1), lambda j, i: (i, 0)),
            pl.BlockSpec((BN, h1), lambda j, i: (i, 0)),
            pl.BlockSpec((BN, h1), lambda j, i: (i, 0)),
            pl.BlockSpec((1, h1, h2), lambda j, i: (j, 0, 0)),
        ],
        out_specs=pl.BlockSpec((1, BN, h2), lambda j, i: (j, i, 0)),
        out_shape=jax.ShapeDtypeStruct((r, N, h2), jnp.float32),
    )(s1, p0, p1, W)


def _self2_body(s_ref, p0_ref, p1_ref, r_ref, b_ref, o_ref):
    h = jnp.maximum(s_ref[...] + p0_ref[...] + p1_ref[...], 0.0)
    o_ref[...] = jnp.dot(h, r_ref[...],
                         preferred_element_type=jnp.float32) + b_ref[...]


def _self2_term(s1, p0, p1, root, b):
    h1, h2 = root.shape
    return pl.pallas_call(
        _self2_body,
        grid=(NB,),
        in_specs=[
            pl.BlockSpec((BN, h1), lambda i: (i, 0)),
            pl.BlockSpec((BN, h1), lambda i: (i, 0)),
            pl.BlockSpec((BN, h1), lambda i: (i, 0)),
            pl.BlockSpec((h1, h2), lambda i: (0, 0)),
            pl.BlockSpec((1, h2), lambda i: (0, 0)),
        ],
        out_specs=pl.BlockSpec((BN, h2), lambda i: (i, 0)),
        out_shape=jax.ShapeDtypeStruct((N, h2), jnp.float32),
    )(s1, p0, p1, root, b.reshape(1, h2))


def _final_body(s_ref, p0_ref, p1_ref, o_ref):
    o_ref[...] = s_ref[...] + p0_ref[...] + p1_ref[...]


def _final_sum(s2, p0, p1):
    h = s2.shape[1]
    return pl.pallas_call(
        _final_body,
        grid=(NB,),
        in_specs=[pl.BlockSpec((BN, h), lambda i: (i, 0))] * 3,
        out_specs=pl.BlockSpec((BN, h), lambda i: (i, 0)),
        out_shape=jax.ShapeDtypeStruct((N, h), jnp.float32),
    )(s2, p0, p1)


def _edge_idx_body(src_ref, dst_ref, rel_ref, g_ref, k_ref):
    rn = rel_ref[...] * N
    g_ref[...] = rn + src_ref[...]
    k_ref[...] = rn + dst_ref[...]


def _edge_indices(src, dst, rel):
    er = E // 128
    bl = pl.BlockSpec((er, 128), lambda i: (0, 0))
    shp = jax.ShapeDtypeStruct((er, 128), jnp.int32)
    g, k = pl.pallas_call(
        _edge_idx_body,
        grid=(1,),
        in_specs=[bl, bl, bl],
        out_specs=[bl, bl],
        out_shape=[shp, shp],
    )(src.reshape(er, 128), dst.reshape(er, 128), rel.reshape(er, 128))
    return g.reshape(E), k.reshape(E)


def _weights_body(p_ref, o_ref):
    c = p_ref[0] + p_ref[1]
    o_ref[...] = 1.0 / jnp.maximum(c, 1.0)


def _weights(partials):
    p = partials.reshape(2, HR * 16 // 128, 128)
    nr = p.shape[1]
    w = pl.pallas_call(
        _weights_body,
        grid=(5,),
        in_specs=[pl.BlockSpec((2, nr // 5, 128), lambda i: (0, i, 0))],
        out_specs=pl.BlockSpec((nr // 5, 128), lambda i: (i, 0)),
        out_shape=jax.ShapeDtypeStruct((nr, 128), jnp.float32),
    )(p)
    return w.reshape(HR * 16)[:RN]


_MESH = plsc.VectorSubcoreMesh(core_axis_name="c", subcore_axis_name="s")
_SC_PARAMS = pltpu.CompilerParams(needs_layout_passes=False,
                                  use_tc_tiling_on_sc=False)


def _hist_body(key_hbm, out_hbm, keyv, rowv, block, shared):
    cid = lax.axis_index("c")
    sid = lax.axis_index("s")
    wid = cid * 16 + sid
    zero16 = jnp.zeros((16,), jnp.float32)
    iota16 = lax.iota(jnp.int32, 16)

    def zb(j, _):
        block[j, :] = zero16
        return 0
    lax.fori_loop(0, CH, zb, 0)
    for t in range(4):
        pltpu.sync_copy(block, shared.at[pl.ds(sid * 320 + t * CH, CH)])
    plsc.subcore_barrier()

    def chunk(i, _):
        base = wid * EPT + i * CH
        pltpu.sync_copy(key_hbm.at[pl.ds(base, CH)], keyv)
        for s in range(CH // 16):
            k16 = keyv[pl.ds(s * 16, 16)]
            rowv[pl.ds(s * 16, 16)] = lax.shift_right_logical(k16, 4)
            col16 = k16 & 15
            for l in range(16):
                block[s * 16 + l, :] = jnp.where(
                    iota16 == col16[l], 1.0, 0.0)
        pltpu.sync_copy(block, shared.at[rowv], add=True)
        return 0
    lax.fori_loop(0, NG, chunk, 0)

    plsc.subcore_barrier()
    pltpu.sync_copy(shared.at[pl.ds(sid * 320, 320)],
                    out_hbm.at[cid, pl.ds(sid * 320, 320)])


@functools.partial(
    pl.kernel,
    out_type=jax.ShapeDtypeStruct((2, HR, 16), jnp.float32),
    mesh=_MESH,
    compiler_params=_SC_PARAMS,
    scratch_types=[
        pltpu.VMEM((CH,), jnp.int32),
        pltpu.VMEM((CH,), jnp.int32),
        pltpu.VMEM((CH, 16), jnp.float32),
        pltpu.VMEM_SHARED((HR, 16), jnp.float32),
    ],
)
def _hist(key_hbm, out_hbm, keyv, rowv, block, shared):
    _hist_body(key_hbm, out_hbm, keyv, rowv, block, shared)


def _make_layer(h):
    nrows = 624

    def body(g_hbm, key_hbm, dst_hbm, w_hbm, xw_hbm, out_hbm,
             gv, keyv, dstv, wev, rows, shared, sem, wsem):
        cid = lax.axis_index("c")
        sid = lax.axis_index("s")
        wid = cid * 16 + sid
        zero16 = jnp.zeros((16,), jnp.float32)

        def zb(j, _):
            for c in range(h // 16):
                rows[j, pl.ds(c * 16, 16)] = zero16
            return 0
        lax.fori_loop(0, CH, zb, 0)
        for t in range(nrows // CH):
            pltpu.sync_copy(rows, shared.at[pl.ds(sid * nrows + t * CH, CH)])
        rem = nrows % CH
        if rem:
            pltpu.sync_copy(
                rows.at[pl.ds(0, rem)],
                shared.at[pl.ds(sid * nrows + (nrows // CH) * CH, rem)])
        tail = N - 16 * nrows
        if tail:
            @pl.when(sid == 15)
            def _():
                pltpu.sync_copy(rows.at[pl.ds(0, tail)],
                                shared.at[pl.ds(16 * nrows, tail)])
        plsc.subcore_barrier()

        def chunk(i, _):
            base = wid * EPT + i * CH
            pltpu.sync_copy(g_hbm.at[pl.ds(base, CH)], gv)
            pltpu.sync_copy(key_hbm.at[pl.ds(base, CH)], keyv)
            pltpu.sync_copy(dst_hbm.at[pl.ds(base, CH)], dstv)
            cp = pltpu.async_copy(xw_hbm.at[gv], rows, sem)
            cw = pltpu.async_copy(w_hbm.at[keyv], wev, wsem)
            cw.wait()
            cp.wait()
            for s in range(CH // 16):
                w16 = wev[pl.ds(s * 16, 16)]
                for l in range(16):
                    wj = w16[l]
                    j = s * 16 + l
                    for c in range(h // 16):
                        sl = pl.ds(c * 16, 16)
                        rows[j, sl] = rows[j, sl] * wj
            pltpu.sync_copy(rows, shared.at[dstv], add=True)
            return 0
        lax.fori_loop(0, NG, chunk, 0)

        plsc.subcore_barrier()
        pltpu.sync_copy(shared.at[pl.ds(sid * nrows, nrows)],
                        out_hbm.at[cid, pl.ds(sid * nrows, nrows)])
        tail = N - 16 * nrows
        if tail:
            @pl.when(sid == 15)
            def _():
                pltpu.sync_copy(shared.at[pl.ds(16 * nrows, tail)],
                                out_hbm.at[cid, pl.ds(16 * nrows, tail)])

    return pl.kernel(
        body,
        out_type=jax.ShapeDtypeStruct((2, N, h), jnp.float32),
        mesh=_MESH,
        compiler_params=_SC_PARAMS,
        scratch_types=[
            pltpu.VMEM((CH,), jnp.int32),
            pltpu.VMEM((CH,), jnp.int32),
            pltpu.VMEM((CH,), jnp.int32),
            pltpu.VMEM((CH,), jnp.float32),
            pltpu.VMEM((CH, h), jnp.float32),
            pltpu.VMEM_SHARED((N, h), jnp.float32),
            pltpu.SemaphoreType.DMA,
            pltpu.SemaphoreType.DMA,
        ],
    )


_layer1 = _make_layer(64)
_layer2 = _make_layer(128)


def kernel(einds, feats, rels, W1, root1, b1, W2, root2, b2):
    src = einds[0]
    dst = einds[1]
    h1 = W1.shape[2]
    h2 = W2.shape[2]

    g, key = _edge_indices(src, dst, rels)
    partials = _hist(key)
    w = _weights(partials)

    xw1 = _xw_tables(feats, W1).reshape(RN, h1)
    s1 = _self_term(feats, root1, b1)
    p1 = _layer1(g, key, dst, w, xw1)

    xw2 = _xw2_tables(s1, p1[0], p1[1], W2).reshape(RN, h2)
    s2 = _self2_term(s1, p1[0], p1[1], root2, b2)
    p2 = _layer2(g, key, dst, w, xw2)

    return _final_sum(s2, p2[0], p2[1])


# speedup vs baseline: 19.1617x; 1.4194x over previous
import functools

import jax
import jax.numpy as jnp
from jax import lax
from jax.experimental import pallas as pl
from jax.experimental.pallas import tpu as pltpu
from jax.experimental.pallas import tpu_sc as plsc

N = 10000
E = 320000
R = 8
RN = R * N
NT = 32
EPT = E // NT
CH = 80
NG = EPT // CH
HR = 5120
BN = 400
NB = N // BN


def _xw_body(x_ref, w_ref, o_ref):
    o_ref[...] = jnp.dot(x_ref[...], w_ref[0],
                         preferred_element_type=jnp.float32)[None]


def _xw_tables(x, W):
    r, din, h = W.shape
    return pl.pallas_call(
        _xw_body,
        grid=(r, NB),
        in_specs=[
            pl.BlockSpec((BN, din), lambda j, i: (i, 0)),
            pl.BlockSpec((1, din, h), lambda j, i: (j, 0, 0)),
        ],
        out_specs=pl.BlockSpec((1, BN, h), lambda j, i: (j, i, 0)),
        out_shape=jax.ShapeDtypeStruct((r, N, h), jnp.float32),
    )(x, W)


def _self_body(x_ref, r_ref, b_ref, o_ref):
    o_ref[...] = jnp.dot(x_ref[...], r_ref[...],
                         preferred_element_type=jnp.float32) + b_ref[...]


def _self_term(x, root, b):
    din, h = root.shape
    return pl.pallas_call(
        _self_body,
        grid=(NB,),
        in_specs=[
            pl.BlockSpec((BN, din), lambda i: (i, 0)),
            pl.BlockSpec((din, h), lambda i: (0, 0)),
            pl.BlockSpec((1, h), lambda i: (0, 0)),
        ],
        out_specs=pl.BlockSpec((BN, h), lambda i: (i, 0)),
        out_shape=jax.ShapeDtypeStruct((N, h), jnp.float32),
    )(x, root, b.reshape(1, h))


def _xw2_body(s_ref, p0_ref, p1_ref, w_ref, o_ref):
    h = jnp.maximum(s_ref[...] + p0_ref[...] + p1_ref[...], 0.0)
    o_ref[...] = jnp.dot(h, w_ref[0], preferred_element_type=jnp.float32)[None]


def _xw2_tables(s1, p0, p1, W):
    r, h1, h2 = W.shape
    return pl.pallas_call(
        _xw2_body,
        grid=(r, NB),
        in_specs=[
            pl.BlockSpec((BN, h1), lambda j, i: (i, 0)),
            pl.BlockSpec((BN, h1), lambda j, i: (i, 0)),
            pl.BlockSpec((BN, h1), lambda j, i: (i, 0)),
            pl.BlockSpec((1, h1, h2), lambda j, i: (j, 0, 0)),
        ],
        out_specs=pl.BlockSpec((1, BN, h2), lambda j, i: (j, i, 0)),
        out_shape=jax.ShapeDtypeStruct((r, N, h2), jnp.float32),
    )(s1, p0, p1, W)


def _self2_body(s_ref, p0_ref, p1_ref, r_ref, b_ref, o_ref):
    h = jnp.maximum(s_ref[...] + p0_ref[...] + p1_ref[...], 0.0)
    o_ref[...] = jnp.dot(h, r_ref[...],
                         preferred_element_type=jnp.float32) + b_ref[...]


def _self2_term(s1, p0, p1, root, b):
    h1, h2 = root.shape
    return pl.pallas_call(
        _self2_body,
        grid=(NB,),
        in_specs=[
            pl.BlockSpec((BN, h1), lambda i: (i, 0)),
            pl.BlockSpec((BN, h1), lambda i: (i, 0)),
            pl.BlockSpec((BN, h1), lambda i: (i, 0)),
            pl.BlockSpec((h1, h2), lambda i: (0, 0)),
            pl.BlockSpec((1, h2), lambda i: (0, 0)),
        ],
        out_specs=pl.BlockSpec((BN, h2), lambda i: (i, 0)),
        out_shape=jax.ShapeDtypeStruct((N, h2), jnp.float32),
    )(s1, p0, p1, root, b.reshape(1, h2))


def _final_body(s_ref, p0_ref, p1_ref, o_ref):
    o_ref[...] = s_ref[...] + p0_ref[...] + p1_ref[...]


def _final_sum(s2, p0, p1):
    h = s2.shape[1]
    return pl.pallas_call(
        _final_body,
        grid=(NB,),
        in_specs=[pl.BlockSpec((BN, h), lambda i: (i, 0))] * 3,
        out_specs=pl.BlockSpec((BN, h), lambda i: (i, 0)),
        out_shape=jax.ShapeDtypeStruct((N, h), jnp.float32),
    )(s2, p0, p1)


def _edge_idx_body(src_ref, dst_ref, rel_ref, g_ref, k_ref):
    rn = rel_ref[...] * N
    g_ref[...] = rn + src_ref[...]
    k_ref[...] = rn + dst_ref[...]


def _edge_indices(src, dst, rel):
    er = E // 128
    bl = pl.BlockSpec((er, 128), lambda i: (0, 0))
    shp = jax.ShapeDtypeStruct((er, 128), jnp.int32)
    g, k = pl.pallas_call(
        _edge_idx_body,
        grid=(1,),
        in_specs=[bl, bl, bl],
        out_specs=[bl, bl],
        out_shape=[shp, shp],
    )(src.reshape(er, 128), dst.reshape(er, 128), rel.reshape(er, 128))
    return g.reshape(E), k.reshape(E)


def _weights_body(p_ref, o_ref):
    c = p_ref[0] + p_ref[1]
    o_ref[...] = 1.0 / jnp.maximum(c, 1.0)


def _weights(partials):
    p = partials.reshape(2, HR * 16 // 128, 128)
    nr = p.shape[1]
    w = pl.pallas_call(
        _weights_body,
        grid=(5,),
        in_specs=[pl.BlockSpec((2, nr // 5, 128), lambda i: (0, i, 0))],
        out_specs=pl.BlockSpec((nr // 5, 128), lambda i: (i, 0)),
        out_shape=jax.ShapeDtypeStruct((nr, 128), jnp.float32),
    )(p)
    return w.reshape(HR * 16)[:RN]


_MESH = plsc.VectorSubcoreMesh(core_axis_name="c", subcore_axis_name="s")
_SC_PARAMS = pltpu.CompilerParams(needs_layout_passes=False,
                                  use_tc_tiling_on_sc=False)


def _hist_body(key_hbm, out_hbm, keyv, rowv, block, shared):
    cid = lax.axis_index("c")
    sid = lax.axis_index("s")
    wid = cid * 16 + sid
    zero16 = jnp.zeros((16,), jnp.float32)
    iota16 = lax.iota(jnp.int32, 16)

    def zb(j, _):
        block[j, :] = zero16
        return 0
    lax.fori_loop(0, CH, zb, 0)
    for t in range(4):
        pltpu.sync_copy(block, shared.at[pl.ds(sid * 320 + t * CH, CH)])
    plsc.subcore_barrier()

    def chunk(i, _):
        base = wid * EPT + i * CH
        pltpu.sync_copy(key_hbm.at[pl.ds(base, CH)], keyv)
        for s in range(CH // 16):
            k16 = keyv[pl.ds(s * 16, 16)]
            rowv[pl.ds(s * 16, 16)] = lax.shift_right_logical(k16, 4)
            col16 = k16 & 15
            for l in range(16):
                block[s * 16 + l, :] = jnp.where(
                    iota16 == col16[l], 1.0, 0.0)
        pltpu.sync_copy(block, shared.at[rowv], add=True)
        return 0
    lax.fori_loop(0, NG, chunk, 0)

    plsc.subcore_barrier()
    pltpu.sync_copy(shared.at[pl.ds(sid * 320, 320)],
                    out_hbm.at[cid, pl.ds(sid * 320, 320)])


@functools.partial(
    pl.kernel,
    out_type=jax.ShapeDtypeStruct((2, HR, 16), jnp.float32),
    mesh=_MESH,
    compiler_params=_SC_PARAMS,
    scratch_types=[
        pltpu.VMEM((CH,), jnp.int32),
        pltpu.VMEM((CH,), jnp.int32),
        pltpu.VMEM((CH, 16), jnp.float32),
        pltpu.VMEM_SHARED((HR, 16), jnp.float32),
    ],
)
def _hist(key_hbm, out_hbm, keyv, rowv, block, shared):
    _hist_body(key_hbm, out_hbm, keyv, rowv, block, shared)


def _make_layer(h):
    nrows = 624

    def body(g_hbm, key_hbm, dst_hbm, w_hbm, xw_hbm, out_hbm,
             gvb, kvb, dvb, wvb, rows,
             shared, s0, s1, s2, s3, s4, s5, s6, s7, s8, s9):
        cid = lax.axis_index("c")
        sid = lax.axis_index("s")
        wid = cid * 16 + sid
        zero16 = jnp.zeros((16,), jnp.float32)

        def zb(j, _):
            for c in range(h // 16):
                rows[0, j, pl.ds(c * 16, 16)] = zero16
            return 0
        lax.fori_loop(0, CH, zb, 0)
        zsrc = rows.at[0]
        for t in range(nrows // CH):
            pltpu.sync_copy(zsrc, shared.at[pl.ds(sid * nrows + t * CH, CH)])
        rem = nrows % CH
        if rem:
            pltpu.sync_copy(
                zsrc.at[pl.ds(0, rem)],
                shared.at[pl.ds(sid * nrows + (nrows // CH) * CH, rem)])
        tail = N - 16 * nrows
        if tail:
            @pl.when(sid == 15)
            def _():
                pltpu.sync_copy(zsrc.at[pl.ds(0, tail)],
                                shared.at[pl.ds(16 * nrows, tail)])
        plsc.subcore_barrier()

        def scale(p):
            for s_ in range(CH // 16):
                w16 = wvb[p, pl.ds(s_ * 16, 16)]
                for l in range(16):
                    wj = w16[l]
                    j = s_ * 16 + l
                    for c in range(h // 16):
                        sl = pl.ds(c * 16, 16)
                        rows[p, j, sl] = rows[p, j, sl] * wj

        def loads(i, slot, sg, sk, sd):
            cg = pltpu.async_copy(g_hbm.at[wid, i], gvb.at[slot], sg)
            ck = pltpu.async_copy(key_hbm.at[wid, i], kvb.at[slot], sk)
            cd = pltpu.async_copy(dst_hbm.at[wid, i], dvb.at[slot], sd)
            return cg, ck, cd

        def gathers(slot, sr, sw):
            cr = pltpu.async_copy(xw_hbm.at[gvb.at[slot]], rows.at[slot], sr)
            cw = pltpu.async_copy(w_hbm.at[kvb.at[slot]], wvb.at[slot], sw)
            return cr, cw

        def finish(slot, cr, cw, cd):
            cr.wait()
            cw.wait()
            scale(slot)
            cd.wait()
            pltpu.sync_copy(rows.at[slot], shared.at[dvb.at[slot]], add=True)

        def pair(t, _):
            ia = 2 * t
            ag, ak, ad = loads(ia, 0, s0, s1, s2)
            bg, bk, bd = loads(ia + 1, 1, s3, s4, s5)
            ag.wait()
            ak.wait()
            ar, aw = gathers(0, s6, s7)
            bg.wait()
            bk.wait()
            br, bw = gathers(1, s8, s9)
            finish(0, ar, aw, ad)
            finish(1, br, bw, bd)
            return 0
        lax.fori_loop(0, NG // 2, pair, 0)

        if NG % 2:
            ia = NG - 1
            ag, ak, ad = loads(ia, 0, s0, s1, s2)
            ag.wait()
            ak.wait()
            ar, aw = gathers(0, s6, s7)
            finish(0, ar, aw, ad)

        plsc.subcore_barrier()
        pltpu.sync_copy(shared.at[pl.ds(sid * nrows, nrows)],
                        out_hbm.at[cid, pl.ds(sid * nrows, nrows)])
        if tail:
            @pl.when(sid == 15)
            def _():
                pltpu.sync_copy(shared.at[pl.ds(16 * nrows, tail)],
                                out_hbm.at[cid, pl.ds(16 * nrows, tail)])

    return pl.kernel(
        body,
        out_type=jax.ShapeDtypeStruct((2, N, h), jnp.float32),
        mesh=_MESH,
        compiler_params=_SC_PARAMS,
        scratch_types=[
            pltpu.VMEM((2, CH), jnp.int32),
            pltpu.VMEM((2, CH), jnp.int32),
            pltpu.VMEM((2, CH), jnp.int32),
            pltpu.VMEM((2, CH), jnp.float32),
            pltpu.VMEM((2, CH, h), jnp.float32),
            pltpu.VMEM_SHARED((N, h), jnp.float32),
        ] + [pltpu.SemaphoreType.DMA] * 10,
    )


_layer1 = _make_layer(64)
_layer2 = _make_layer(128)


def kernel(einds, feats, rels, W1, root1, b1, W2, root2, b2):
    src = einds[0]
    dst = einds[1]
    h1 = W1.shape[2]
    h2 = W2.shape[2]

    g, key = _edge_indices(src, dst, rels)
    partials = _hist(key)
    w = _weights(partials)

    g3 = g.reshape(NT, NG, CH)
    k3 = key.reshape(NT, NG, CH)
    d3 = dst.reshape(NT, NG, CH)

    xw1 = _xw_tables(feats, W1).reshape(RN, h1)
    s1 = _self_term(feats, root1, b1)
    p1 = _layer1(g3, k3, d3, w, xw1)

    xw2 = _xw2_tables(s1, p1[0], p1[1], W2).reshape(RN, h2)
    s2 = _self2_term(s1, p1[0], p1[1], root2, b2)
    p2 = _layer2(g3, k3, d3, w, xw2)

    return _final_sum(s2, p2[0], p2[1])


# speedup vs baseline: 19.2271x; 1.0034x over previous
import functools

import jax
import jax.numpy as jnp
from jax import lax
from jax.experimental import pallas as pl
from jax.experimental.pallas import tpu as pltpu
from jax.experimental.pallas import tpu_sc as plsc

N = 10000
E = 320000
R = 8
RN = R * N
NT = 32
EPT = E // NT
CH = 80
NG = EPT // CH
HR = 5120
BN = 400
NB = N // BN


def _xw_body(x_ref, w_ref, o_ref):
    o_ref[...] = jnp.dot(x_ref[...], w_ref[0],
                         preferred_element_type=jnp.float32)[None]


def _xw_tables(x, W):
    r, din, h = W.shape
    return pl.pallas_call(
        _xw_body,
        grid=(r, NB),
        in_specs=[
            pl.BlockSpec((BN, din), lambda j, i: (i, 0)),
            pl.BlockSpec((1, din, h), lambda j, i: (j, 0, 0)),
        ],
        out_specs=pl.BlockSpec((1, BN, h), lambda j, i: (j, i, 0)),
        out_shape=jax.ShapeDtypeStruct((r, N, h), jnp.float32),
    )(x, W)


def _self_body(x_ref, r_ref, b_ref, o_ref):
    o_ref[...] = jnp.dot(x_ref[...], r_ref[...],
                         preferred_element_type=jnp.float32) + b_ref[...]


def _self_term(x, root, b):
    din, h = root.shape
    return pl.pallas_call(
        _self_body,
        grid=(NB,),
        in_specs=[
            pl.BlockSpec((BN, din), lambda i: (i, 0)),
            pl.BlockSpec((din, h), lambda i: (0, 0)),
            pl.BlockSpec((1, h), lambda i: (0, 0)),
        ],
        out_specs=pl.BlockSpec((BN, h), lambda i: (i, 0)),
        out_shape=jax.ShapeDtypeStruct((N, h), jnp.float32),
    )(x, root, b.reshape(1, h))


def _xw2_body(s_ref, p0_ref, p1_ref, w_ref, o_ref):
    h = jnp.maximum(s_ref[...] + p0_ref[...] + p1_ref[...], 0.0)
    o_ref[...] = jnp.dot(h, w_ref[0], preferred_element_type=jnp.float32)[None]


def _xw2_tables(s1, p0, p1, W):
    r, h1, h2 = W.shape
    return pl.pallas_call(
        _xw2_body,
        grid=(r, NB),
        in_specs=[
            pl.BlockSpec((BN, h1), lambda j, i: (i, 0)),
            pl.BlockSpec((BN, h1), lambda j, i: (i, 0)),
            pl.BlockSpec((BN, h1), lambda j, i: (i, 0)),
            pl.BlockSpec((1, h1, h2), lambda j, i: (j, 0, 0)),
        ],
        out_specs=pl.BlockSpec((1, BN, h2), lambda j, i: (j, i, 0)),
        out_shape=jax.ShapeDtypeStruct((r, N, h2), jnp.float32),
    )(s1, p0, p1, W)


def _self2_body(s_ref, p0_ref, p1_ref, r_ref, b_ref, o_ref):
    h = jnp.maximum(s_ref[...] + p0_ref[...] + p1_ref[...], 0.0)
    o_ref[...] = jnp.dot(h, r_ref[...],
                         preferred_element_type=jnp.float32) + b_ref[...]


def _self2_term(s1, p0, p1, root, b):
    h1, h2 = root.shape
    return pl.pallas_call(
        _self2_body,
        grid=(NB,),
        in_specs=[
            pl.BlockSpec((BN, h1), lambda i: (i, 0)),
            pl.BlockSpec((BN, h1), lambda i: (i, 0)),
            pl.BlockSpec((BN, h1), lambda i: (i, 0)),
            pl.BlockSpec((h1, h2), lambda i: (0, 0)),
            pl.BlockSpec((1, h2), lambda i: (0, 0)),
        ],
        out_specs=pl.BlockSpec((BN, h2), lambda i: (i, 0)),
        out_shape=jax.ShapeDtypeStruct((N, h2), jnp.float32),
    )(s1, p0, p1, root, b.reshape(1, h2))


def _final_body(s_ref, p0_ref, p1_ref, o_ref):
    o_ref[...] = s_ref[...] + p0_ref[...] + p1_ref[...]


def _final_sum(s2, p0, p1):
    h = s2.shape[1]
    return pl.pallas_call(
        _final_body,
        grid=(NB,),
        in_specs=[pl.BlockSpec((BN, h), lambda i: (i, 0))] * 3,
        out_specs=pl.BlockSpec((BN, h), lambda i: (i, 0)),
        out_shape=jax.ShapeDtypeStruct((N, h), jnp.float32),
    )(s2, p0, p1)


def _edge_idx_body(src_ref, dst_ref, rel_ref, g_ref, k_ref):
    rn = rel_ref[...] * N
    g_ref[...] = rn + src_ref[...]
    k_ref[...] = rn + dst_ref[...]


def _edge_indices(src, dst, rel):
    er = E // 128
    bl = pl.BlockSpec((er, 128), lambda i: (0, 0))
    shp = jax.ShapeDtypeStruct((er, 128), jnp.int32)
    g, k = pl.pallas_call(
        _edge_idx_body,
        grid=(1,),
        in_specs=[bl, bl, bl],
        out_specs=[bl, bl],
        out_shape=[shp, shp],
    )(src.reshape(er, 128), dst.reshape(er, 128), rel.reshape(er, 128))
    return g.reshape(E), k.reshape(E)


def _weights_body(p_ref, o_ref):
    c = p_ref[0] + p_ref[1]
    o_ref[...] = 1.0 / jnp.maximum(c, 1.0)


def _weights(partials):
    p = partials.reshape(2, HR * 16 // 128, 128)
    nr = p.shape[1]
    w = pl.pallas_call(
        _weights_body,
        grid=(5,),
        in_specs=[pl.BlockSpec((2, nr // 5, 128), lambda i: (0, i, 0))],
        out_specs=pl.BlockSpec((nr // 5, 128), lambda i: (i, 0)),
        out_shape=jax.ShapeDtypeStruct((nr, 128), jnp.float32),
    )(p)
    return w.reshape(HR * 16)[:RN]


_MESH = plsc.VectorSubcoreMesh(core_axis_name="c", subcore_axis_name="s")
_SC_PARAMS = pltpu.CompilerParams(needs_layout_passes=False,
                                  use_tc_tiling_on_sc=False)


def _hist_body(key_hbm, out_hbm, keyv, rowv, block, shared):
    cid = lax.axis_index("c")
    sid = lax.axis_index("s")
    wid = cid * 16 + sid
    zero16 = jnp.zeros((16,), jnp.float32)
    iota16 = lax.iota(jnp.int32, 16)

    def zb(j, _):
        block[j, :] = zero16
        return 0
    lax.fori_loop(0, CH, zb, 0)
    for t in range(4):
        pltpu.sync_copy(block, shared.at[pl.ds(sid * 320 + t * CH, CH)])
    plsc.subcore_barrier()

    def chunk(i, _):
        base = wid * EPT + i * CH
        pltpu.sync_copy(key_hbm.at[pl.ds(base, CH)], keyv)
        for s in range(CH // 16):
            k16 = keyv[pl.ds(s * 16, 16)]
            rowv[pl.ds(s * 16, 16)] = lax.shift_right_logical(k16, 4)
            col16 = k16 & 15
            for l in range(16):
                block[s * 16 + l, :] = jnp.where(
                    iota16 == col16[l], 1.0, 0.0)
        pltpu.sync_copy(block, shared.at[rowv], add=True)
        return 0
    lax.fori_loop(0, NG, chunk, 0)

    plsc.subcore_barrier()
    pltpu.sync_copy(shared.at[pl.ds(sid * 320, 320)],
                    out_hbm.at[cid, pl.ds(sid * 320, 320)])


@functools.partial(
    pl.kernel,
    out_type=jax.ShapeDtypeStruct((2, HR, 16), jnp.float32),
    mesh=_MESH,
    compiler_params=_SC_PARAMS,
    scratch_types=[
        pltpu.VMEM((CH,), jnp.int32),
        pltpu.VMEM((CH,), jnp.int32),
        pltpu.VMEM((CH, 16), jnp.float32),
        pltpu.VMEM_SHARED((HR, 16), jnp.float32),
    ],
)
def _hist(key_hbm, out_hbm, keyv, rowv, block, shared):
    _hist_body(key_hbm, out_hbm, keyv, rowv, block, shared)


def _make_layer(h):
    nrows = 624

    def body(g_hbm, key_hbm, dst_hbm, w_hbm, xw_hbm, out_hbm,
             gvb, kvb, dvb, wvb, rows,
             shared, sgl, skl, sdl, srg, swg, ssc):
        cid = lax.axis_index("c")
        sid = lax.axis_index("s")
        wid = cid * 16 + sid
        zero16 = jnp.zeros((16,), jnp.float32)

        def zb(j, _):
            for c in range(h // 16):
                rows[0, j, pl.ds(c * 16, 16)] = zero16
            return 0
        lax.fori_loop(0, CH, zb, 0)
        zsrc = rows.at[0]
        for t in range(nrows // CH):
            pltpu.sync_copy(zsrc, shared.at[pl.ds(sid * nrows + t * CH, CH)])
        rem = nrows % CH
        if rem:
            pltpu.sync_copy(
                zsrc.at[pl.ds(0, rem)],
                shared.at[pl.ds(sid * nrows + (nrows // CH) * CH, rem)])
        tail = N - 16 * nrows
        if tail:
            @pl.when(sid == 15)
            def _():
                pltpu.sync_copy(zsrc.at[pl.ds(0, tail)],
                                shared.at[pl.ds(16 * nrows, tail)])
        plsc.subcore_barrier()

        def scale(p):
            for s_ in range(CH // 16):
                w16 = wvb[p, pl.ds(s_ * 16, 16)]
                for l in range(16):
                    wj = w16[l]
                    j = s_ * 16 + l
                    for c in range(h // 16):
                        sl = pl.ds(c * 16, 16)
                        rows[p, j, sl] = rows[p, j, sl] * wj

        def loads(i, q):
            return (pltpu.async_copy(g_hbm.at[wid, i], gvb.at[q], sgl.at[q]),
                    pltpu.async_copy(key_hbm.at[wid, i], kvb.at[q],
                                     skl.at[q]),
                    pltpu.async_copy(dst_hbm.at[wid, i], dvb.at[q],
                                     sdl.at[q]))

        def gathers(q):
            return (pltpu.async_copy(xw_hbm.at[gvb.at[q]], rows.at[q],
                                     srg.at[q]),
                    pltpu.async_copy(w_hbm.at[kvb.at[q]], wvb.at[q],
                                     swg.at[q]))

        Q = 4

        def quad(t, _):
            ia = Q * t
            ld = [loads(ia + q, q) for q in range(Q)]
            gs = []
            for q in range(Q):
                ld[q][0].wait()
                ld[q][1].wait()
                gs.append(gathers(q))
            scat = []
            for q in range(Q):
                gs[q][0].wait()
                gs[q][1].wait()
                scale(q)
                ld[q][2].wait()
                scat.append(pltpu.async_copy(rows.at[q],
                                             shared.at[dvb.at[q]],
                                             ssc.at[q], add=True))
            for c in scat:
                c.wait()
            return 0
        lax.fori_loop(0, NG // Q, quad, 0)

        for ia in range((NG // Q) * Q, NG):
            ag, ak, ad = loads(ia, 0)
            ag.wait()
            ak.wait()
            ar, aw = gathers(0)
            ar.wait()
            aw.wait()
            scale(0)
            ad.wait()
            pltpu.sync_copy(rows.at[0], shared.at[dvb.at[0]], add=True)

        plsc.subcore_barrier()
        pltpu.sync_copy(shared.at[pl.ds(sid * nrows, nrows)],
                        out_hbm.at[cid, pl.ds(sid * nrows, nrows)])
        if tail:
            @pl.when(sid == 15)
            def _():
                pltpu.sync_copy(shared.at[pl.ds(16 * nrows, tail)],
                                out_hbm.at[cid, pl.ds(16 * nrows, tail)])

    return pl.kernel(
        body,
        out_type=jax.ShapeDtypeStruct((2, N, h), jnp.float32),
        mesh=_MESH,
        compiler_params=_SC_PARAMS,
        scratch_types=[
            pltpu.VMEM((4, CH), jnp.int32),
            pltpu.VMEM((4, CH), jnp.int32),
            pltpu.VMEM((4, CH), jnp.int32),
            pltpu.VMEM((4, CH), jnp.float32),
            pltpu.VMEM((4, CH, h), jnp.float32),
            pltpu.VMEM_SHARED((N, h), jnp.float32),
        ] + [pltpu.SemaphoreType.DMA((4,))] * 6,
    )


_layer1 = _make_layer(64)
_layer2 = _make_layer(128)


def kernel(einds, feats, rels, W1, root1, b1, W2, root2, b2):
    src = einds[0]
    dst = einds[1]
    h1 = W1.shape[2]
    h2 = W2.shape[2]

    g, key = _edge_indices(src, dst, rels)
    partials = _hist(key)
    w = _weights(partials)

    g3 = g.reshape(NT, NG, CH)
    k3 = key.reshape(NT, NG, CH)
    d3 = dst.reshape(NT, NG, CH)

    xw1 = _xw_tables(feats, W1).reshape(RN, h1)
    s1 = _self_term(feats, root1, b1)
    p1 = _layer1(g3, k3, d3, w, xw1)

    xw2 = _xw2_tables(s1, p1[0], p1[1], W2).reshape(RN, h2)
    s2 = _self2_term(s1, p1[0], p1[1], root2, b2)
    p2 = _layer2(g3, k3, d3, w, xw2)

    return _final_sum(s2, p2[0], p2[1])


# speedup vs baseline: 19.2557x; 1.0015x over previous
import functools

import jax
import jax.numpy as jnp
from jax import lax
from jax.experimental import pallas as pl
from jax.experimental.pallas import tpu as pltpu
from jax.experimental.pallas import tpu_sc as plsc

N = 10000
E = 320000
R = 8
RN = R * N
NT = 32
EPT = E // NT
CH = 80
NG = EPT // CH
HR = 5120
BN = 400
NB = N // BN


def _xw_body(x_ref, w_ref, o_ref):
    o_ref[...] = jnp.dot(x_ref[...], w_ref[0],
                         preferred_element_type=jnp.float32)[None]


def _xw_tables(x, W):
    r, din, h = W.shape
    return pl.pallas_call(
        _xw_body,
        grid=(r, NB),
        in_specs=[
            pl.BlockSpec((BN, din), lambda j, i: (i, 0)),
            pl.BlockSpec((1, din, h), lambda j, i: (j, 0, 0)),
        ],
        out_specs=pl.BlockSpec((1, BN, h), lambda j, i: (j, i, 0)),
        out_shape=jax.ShapeDtypeStruct((r, N, h), jnp.float32),
    )(x, W)


def _self_body(x_ref, r_ref, b_ref, o_ref):
    o_ref[...] = jnp.dot(x_ref[...], r_ref[...],
                         preferred_element_type=jnp.float32) + b_ref[...]


def _self_term(x, root, b):
    din, h = root.shape
    return pl.pallas_call(
        _self_body,
        grid=(NB,),
        in_specs=[
            pl.BlockSpec((BN, din), lambda i: (i, 0)),
            pl.BlockSpec((din, h), lambda i: (0, 0)),
            pl.BlockSpec((1, h), lambda i: (0, 0)),
        ],
        out_specs=pl.BlockSpec((BN, h), lambda i: (i, 0)),
        out_shape=jax.ShapeDtypeStruct((N, h), jnp.float32),
    )(x, root, b.reshape(1, h))


def _xw2_body(s_ref, p0_ref, p1_ref, w_ref, o_ref):
    h = jnp.maximum(s_ref[...] + p0_ref[...] + p1_ref[...], 0.0)
    o_ref[...] = jnp.dot(h, w_ref[0], preferred_element_type=jnp.float32)[None]


def _xw2_tables(s1, p0, p1, W):
    r, h1, h2 = W.shape
    return pl.pallas_call(
        _xw2_body,
        grid=(r, NB),
        in_specs=[
            pl.BlockSpec((BN, h1), lambda j, i: (i, 0)),
            pl.BlockSpec((BN, h1), lambda j, i: (i, 0)),
            pl.BlockSpec((BN, h1), lambda j, i: (i, 0)),
            pl.BlockSpec((1, h1, h2), lambda j, i: (j, 0, 0)),
        ],
        out_specs=pl.BlockSpec((1, BN, h2), lambda j, i: (j, i, 0)),
        out_shape=jax.ShapeDtypeStruct((r, N, h2), jnp.float32),
    )(s1, p0, p1, W)


def _self2_body(s_ref, p0_ref, p1_ref, r_ref, b_ref, o_ref):
    h = jnp.maximum(s_ref[...] + p0_ref[...] + p1_ref[...], 0.0)
    o_ref[...] = jnp.dot(h, r_ref[...],
                         preferred_element_type=jnp.float32) + b_ref[...]


def _self2_term(s1, p0, p1, root, b):
    h1, h2 = root.shape
    return pl.pallas_call(
        _self2_body,
        grid=(NB,),
        in_specs=[
            pl.BlockSpec((BN, h1), lambda i: (i, 0)),
            pl.BlockSpec((BN, h1), lambda i: (i, 0)),
            pl.BlockSpec((BN, h1), lambda i: (i, 0)),
            pl.BlockSpec((h1, h2), lambda i: (0, 0)),
            pl.BlockSpec((1, h2), lambda i: (0, 0)),
        ],
        out_specs=pl.BlockSpec((BN, h2), lambda i: (i, 0)),
        out_shape=jax.ShapeDtypeStruct((N, h2), jnp.float32),
    )(s1, p0, p1, root, b.reshape(1, h2))


def _final_body(s_ref, p0_ref, p1_ref, o_ref):
    o_ref[...] = s_ref[...] + p0_ref[...] + p1_ref[...]


def _final_sum(s2, p0, p1):
    h = s2.shape[1]
    return pl.pallas_call(
        _final_body,
        grid=(NB,),
        in_specs=[pl.BlockSpec((BN, h), lambda i: (i, 0))] * 3,
        out_specs=pl.BlockSpec((BN, h), lambda i: (i, 0)),
        out_shape=jax.ShapeDtypeStruct((N, h), jnp.float32),
    )(s2, p0, p1)


def _edge_idx_body(src_ref, dst_ref, rel_ref, g_ref, k_ref):
    rn = rel_ref[...] * N
    g_ref[...] = rn + src_ref[...]
    k_ref[...] = rn + dst_ref[...]


def _edge_indices(src, dst, rel):
    er = E // 128
    bl = pl.BlockSpec((er, 128), lambda i: (0, 0))
    shp = jax.ShapeDtypeStruct((er, 128), jnp.int32)
    g, k = pl.pallas_call(
        _edge_idx_body,
        grid=(1,),
        in_specs=[bl, bl, bl],
        out_specs=[bl, bl],
        out_shape=[shp, shp],
    )(src.reshape(er, 128), dst.reshape(er, 128), rel.reshape(er, 128))
    return g.reshape(E), k.reshape(E)


def _weights_body(p_ref, o_ref):
    c = p_ref[0] + p_ref[1]
    o_ref[...] = 1.0 / jnp.maximum(c, 1.0)


def _weights(partials):
    p = partials.reshape(2, HR * 16 // 128, 128)
    nr = p.shape[1]
    w = pl.pallas_call(
        _weights_body,
        grid=(5,),
        in_specs=[pl.BlockSpec((2, nr // 5, 128), lambda i: (0, i, 0))],
        out_specs=pl.BlockSpec((nr // 5, 128), lambda i: (i, 0)),
        out_shape=jax.ShapeDtypeStruct((nr, 128), jnp.float32),
    )(p)
    return w.reshape(HR * 16)[:RN]


_MESH = plsc.VectorSubcoreMesh(core_axis_name="c", subcore_axis_name="s")
_SC_PARAMS = pltpu.CompilerParams(needs_layout_passes=False,
                                  use_tc_tiling_on_sc=False)


def _hist_body(key_hbm, out_hbm, keyv, rowv, block, shared,
               sl0, sl1, ss0, ss1):
    cid = lax.axis_index("c")
    sid = lax.axis_index("s")
    wid = cid * 16 + sid
    zero16 = jnp.zeros((16,), jnp.float32)
    iota16 = lax.iota(jnp.int32, 16)

    def zb(j, _):
        block[0, j, :] = zero16
        return 0
    lax.fori_loop(0, CH, zb, 0)
    for t in range(4):
        pltpu.sync_copy(block.at[0],
                        shared.at[pl.ds(sid * 320 + t * CH, CH)])
    plsc.subcore_barrier()

    def build(q):
        for s in range(CH // 16):
            k16 = keyv[q, pl.ds(s * 16, 16)]
            rowv[q, pl.ds(s * 16, 16)] = lax.shift_right_logical(k16, 4)
            col16 = k16 & 15
            for l in range(16):
                block[q, s * 16 + l, :] = jnp.where(
                    iota16 == col16[l], 1.0, 0.0)

    def pair(t, _):
        base = wid * EPT + 2 * t * CH
        ca = pltpu.async_copy(key_hbm.at[pl.ds(base, CH)], keyv.at[0], sl0)
        cb = pltpu.async_copy(key_hbm.at[pl.ds(base + CH, CH)], keyv.at[1],
                              sl1)
        ca.wait()
        build(0)
        sa = pltpu.async_copy(block.at[0], shared.at[rowv.at[0]], ss0,
                              add=True)
        cb.wait()
        build(1)
        sb = pltpu.async_copy(block.at[1], shared.at[rowv.at[1]], ss1,
                              add=True)
        sa.wait()
        sb.wait()
        return 0
    lax.fori_loop(0, NG // 2, pair, 0)

    for i in range((NG // 2) * 2, NG):
        base = wid * EPT + i * CH
        pltpu.sync_copy(key_hbm.at[pl.ds(base, CH)], keyv.at[0])
        build(0)
        pltpu.sync_copy(block.at[0], shared.at[rowv.at[0]], add=True)

    plsc.subcore_barrier()
    pltpu.sync_copy(shared.at[pl.ds(sid * 320, 320)],
                    out_hbm.at[cid, pl.ds(sid * 320, 320)])


@functools.partial(
    pl.kernel,
    out_type=jax.ShapeDtypeStruct((2, HR, 16), jnp.float32),
    mesh=_MESH,
    compiler_params=_SC_PARAMS,
    scratch_types=[
        pltpu.VMEM((2, CH), jnp.int32),
        pltpu.VMEM((2, CH), jnp.int32),
        pltpu.VMEM((2, CH, 16), jnp.float32),
        pltpu.VMEM_SHARED((HR, 16), jnp.float32),
        pltpu.SemaphoreType.DMA,
        pltpu.SemaphoreType.DMA,
        pltpu.SemaphoreType.DMA,
        pltpu.SemaphoreType.DMA,
    ],
)
def _hist(key_hbm, out_hbm, keyv, rowv, block, shared, sl0, sl1, ss0, ss1):
    _hist_body(key_hbm, out_hbm, keyv, rowv, block, shared,
               sl0, sl1, ss0, ss1)


def _make_layer(h):
    nrows = 624

    def body(g_hbm, key_hbm, dst_hbm, w_hbm, xw_hbm, out_hbm,
             gvb, kvb, dvb, wvb, rows,
             shared, sgl, skl, sdl, srg, swg, ssc):
        cid = lax.axis_index("c")
        sid = lax.axis_index("s")
        wid = cid * 16 + sid
        zero16 = jnp.zeros((16,), jnp.float32)

        def zb(j, _):
            for c in range(h // 16):
                rows[0, j, pl.ds(c * 16, 16)] = zero16
            return 0
        lax.fori_loop(0, CH, zb, 0)
        zsrc = rows.at[0]
        for t in range(nrows // CH):
            pltpu.sync_copy(zsrc, shared.at[pl.ds(sid * nrows + t * CH, CH)])
        rem = nrows % CH
        if rem:
            pltpu.sync_copy(
                zsrc.at[pl.ds(0, rem)],
                shared.at[pl.ds(sid * nrows + (nrows // CH) * CH, rem)])
        tail = N - 16 * nrows
        if tail:
            @pl.when(sid == 15)
            def _():
                pltpu.sync_copy(zsrc.at[pl.ds(0, tail)],
                                shared.at[pl.ds(16 * nrows, tail)])
        plsc.subcore_barrier()

        def scale(p):
            for s_ in range(CH // 16):
                w16 = wvb[p, pl.ds(s_ * 16, 16)]
                for l in range(16):
                    wj = w16[l]
                    j = s_ * 16 + l
                    for c in range(h // 16):
                        sl = pl.ds(c * 16, 16)
                        rows[p, j, sl] = rows[p, j, sl] * wj

        def loads(i, q):
            return (pltpu.async_copy(g_hbm.at[wid, i], gvb.at[q], sgl.at[q]),
                    pltpu.async_copy(key_hbm.at[wid, i], kvb.at[q],
                                     skl.at[q]),
                    pltpu.async_copy(dst_hbm.at[wid, i], dvb.at[q],
                                     sdl.at[q]))

        def gathers(q):
            return (pltpu.async_copy(xw_hbm.at[gvb.at[q]], rows.at[q],
                                     srg.at[q]),
                    pltpu.async_copy(w_hbm.at[kvb.at[q]], wvb.at[q],
                                     swg.at[q]))

        Q = 4

        def quad(t, _):
            ia = Q * t
            ld = [loads(ia + q, q) for q in range(Q)]
            gs = []
            for q in range(Q):
                ld[q][0].wait()
                ld[q][1].wait()
                gs.append(gathers(q))
            scat = []
            for q in range(Q):
                gs[q][0].wait()
                gs[q][1].wait()
                scale(q)
                ld[q][2].wait()
                scat.append(pltpu.async_copy(rows.at[q],
                                             shared.at[dvb.at[q]],
                                             ssc.at[q], add=True))
            for c in scat:
                c.wait()
            return 0
        lax.fori_loop(0, NG // Q, quad, 0)

        for ia in range((NG // Q) * Q, NG):
            ag, ak, ad = loads(ia, 0)
            ag.wait()
            ak.wait()
            ar, aw = gathers(0)
            ar.wait()
            aw.wait()
            scale(0)
            ad.wait()
            pltpu.sync_copy(rows.at[0], shared.at[dvb.at[0]], add=True)

        plsc.subcore_barrier()
        pltpu.sync_copy(shared.at[pl.ds(sid * nrows, nrows)],
                        out_hbm.at[cid, pl.ds(sid * nrows, nrows)])
        if tail:
            @pl.when(sid == 15)
            def _():
                pltpu.sync_copy(shared.at[pl.ds(16 * nrows, tail)],
                                out_hbm.at[cid, pl.ds(16 * nrows, tail)])

    return pl.kernel(
        body,
        out_type=jax.ShapeDtypeStruct((2, N, h), jnp.float32),
        mesh=_MESH,
        compiler_params=_SC_PARAMS,
        scratch_types=[
            pltpu.VMEM((4, CH), jnp.int32),
            pltpu.VMEM((4, CH), jnp.int32),
            pltpu.VMEM((4, CH), jnp.int32),
            pltpu.VMEM((4, CH), jnp.float32),
            pltpu.VMEM((4, CH, h), jnp.float32),
            pltpu.VMEM_SHARED((N, h), jnp.float32),
        ] + [pltpu.SemaphoreType.DMA((4,))] * 6,
    )


_layer1 = _make_layer(64)
_layer2 = _make_layer(128)


def kernel(einds, feats, rels, W1, root1, b1, W2, root2, b2):
    src = einds[0]
    dst = einds[1]
    h1 = W1.shape[2]
    h2 = W2.shape[2]

    g, key = _edge_indices(src, dst, rels)
    partials = _hist(key)
    w = _weights(partials)

    g3 = g.reshape(NT, NG, CH)
    k3 = key.reshape(NT, NG, CH)
    d3 = dst.reshape(NT, NG, CH)

    xw1 = _xw_tables(feats, W1).reshape(RN, h1)
    s1 = _self_term(feats, root1, b1)
    p1 = _layer1(g3, k3, d3, w, xw1)

    xw2 = _xw2_tables(s1, p1[0], p1[1], W2).reshape(RN, h2)
    s2 = _self2_term(s1, p1[0], p1[1], root2, b2)
    p2 = _layer2(g3, k3, d3, w, xw2)

    return _final_sum(s2, p2[0], p2[1])
